# baseline (speedup 1.0000x reference)
import functools

import jax
import jax.numpy as jnp
from jax import lax
from jax.experimental import pallas as pl
from jax.experimental.pallas import tpu as pltpu
from jax.experimental.pallas import tpu_sc as plsc

_C_BLK = 64


def _sc_lookup(emb_table, F):
    C = emb_table.shape[1]
    mesh = plsc.ScalarSubcoreMesh(axis_name="c", num_cores=1)

    @functools.partial(
        pl.kernel,
        mesh=mesh,
        out_type=jax.ShapeDtypeStruct((F, C), jnp.float32),
    )
    def k(emb_hbm, out_hbm):
        pltpu.sync_copy(emb_hbm.at[pl.ds(0, F)], out_hbm)

    return k(emb_table)


def _add_body(x_ref, fe_ref, o_ref):
    j = pl.program_id(1)
    fe = fe_ref[...].T
    fe_half = jnp.where(j == 0, fe[:_C_BLK], fe[_C_BLK:])
    o_ref[...] = x_ref[...] + fe_half[None, :, :, None]


def _add_body_alias(x_ref, fe_ref, prev_ref, o_ref):
    _add_body(x_ref, fe_ref, o_ref)


def kernel(x, emb_table):
    b, c, f, t = x.shape
    femap = _sc_lookup(emb_table, f)

    part = pl.pallas_call(
        _add_body,
        grid=(b - 1, c // _C_BLK),
        in_specs=[
            pl.BlockSpec((1, _C_BLK, f, t), lambda i, j: (i, j, 0, 0)),
            pl.BlockSpec((f, c), lambda i, j: (0, 0)),
        ],
        out_specs=pl.BlockSpec((1, _C_BLK, f, t), lambda i, j: (i, j, 0, 0)),
        out_shape=jax.ShapeDtypeStruct(x.shape, x.dtype),
    )(x, emb_table[:f])

    return pl.pallas_call(
        _add_body_alias,
        grid=(1, c // _C_BLK),
        in_specs=[
            pl.BlockSpec((1, _C_BLK, f, t), lambda i, j: (b - 1, j, 0, 0)),
            pl.BlockSpec((f, c), lambda i, j: (0, 0)),
            pl.BlockSpec(memory_space=pl.ANY),
        ],
        out_specs=pl.BlockSpec((1, _C_BLK, f, t), lambda i, j: (b - 1, j, 0, 0)),
        out_shape=jax.ShapeDtypeStruct(x.shape, x.dtype),
        input_output_aliases={2: 0},
    )(x, femap, part)
